# baseline (speedup 1.0000x reference)
import functools

import jax
import jax.numpy as jnp
from jax.experimental import pallas as pl
from jax.experimental.pallas import tpu as pltpu

_NUM_Q = 4
_K = 1024
_D = 64
_COMMIT = 0.25
_TILE = 4096
_CHAINS = 4

_PREC = jax.lax.Precision.DEFAULT
_PREC_WSQ = jax.lax.Precision.HIGHEST


def _rvq_body(z_ref, cb_ref, zrec_ref, i0_ref, i1_ref, i2_ref, i3_ref,
              loss_ref, wsq_ref):
    i = pl.program_id(0)

    @pl.when(i == 0)
    def _():
        for q in range(_NUM_Q):
            w = cb_ref[q]
            wsq_ref[q] = jax.lax.dot_general(
                jnp.ones((1, _D), jnp.float32), w * w,
                (((1,), (1,)), ((), ())), precision=_PREC_WSQ)

    z = z_ref[...]
    t = z.shape[0]
    nc = _CHAINS
    h = t // nc
    idx_refs = (i0_ref, i1_ref, i2_ref, i3_ref)
    lane_iota = jax.lax.broadcasted_iota(jnp.int32, (h, _K), 1)
    res = [z[c * h:(c + 1) * h] for c in range(nc)]
    zrec = [jnp.zeros_like(r) for r in res]
    loss_sum = jnp.float32(0.0)
    for q in range(_NUM_Q):
        w = cb_ref[q]
        wsq = wsq_ref[q]
        zw = [jax.lax.dot_general(res[c], w, (((1,), (1,)), ((), ())),
                                  precision=_PREC) for c in range(nc)]
        idx = [None] * nc
        one_hot = [None] * nc
        for c in range(nc):
            zsq = jnp.sum(res[c] * res[c], axis=1, keepdims=True)
            scores = (zsq + wsq) - 2.0 * zw[c]
            smin = jnp.min(scores, axis=1, keepdims=True)
            idx[c] = jnp.min(jnp.where(scores == smin, lane_iota, _K),
                             axis=1, keepdims=True)
            one_hot[c] = (lane_iota == idx[c]).astype(jnp.float32)
        idx_refs[q][...] = jax.lax.concatenate(idx, 0)
        for c in range(nc):
            zq = jax.lax.dot_general(
                one_hot[c], w, (((1,), (0,)), ((), ())),
                precision=_PREC)
            step = zq - res[c]
            zq_ste = res[c] + step
            loss_sum = loss_sum + jnp.sum(step * step)
            zrec[c] = zrec[c] + zq_ste
            res[c] = res[c] - zq_ste
    zrec_ref[...] = jax.lax.concatenate(zrec, 0)

    @pl.when(i == 0)
    def _():
        loss_ref[0, 0] = loss_sum

    @pl.when(i > 0)
    def _():
        loss_ref[0, 0] = loss_ref[0, 0] + loss_sum


@functools.partial(jax.jit, static_argnames=("interpret",))
def kernel(z, codebooks, interpret=False):
    n, d = z.shape
    grid = (n // _TILE,)
    idx_spec = pl.BlockSpec((_TILE, 1), lambda i: (i, 0))
    zrec, i0, i1, i2, i3, loss = pl.pallas_call(
        _rvq_body,
        grid=grid,
        in_specs=[
            pl.BlockSpec((_TILE, d), lambda i: (i, 0)),
            pl.BlockSpec((_NUM_Q, _K, _D), lambda i: (0, 0, 0)),
        ],
        out_specs=[
            pl.BlockSpec((_TILE, d), lambda i: (i, 0)),
            idx_spec, idx_spec, idx_spec, idx_spec,
            pl.BlockSpec((1, 1), lambda i: (0, 0),
                         memory_space=pltpu.SMEM),
        ],
        out_shape=[
            jax.ShapeDtypeStruct((n, d), jnp.float32),
            jax.ShapeDtypeStruct((n, 1), jnp.int32),
            jax.ShapeDtypeStruct((n, 1), jnp.int32),
            jax.ShapeDtypeStruct((n, 1), jnp.int32),
            jax.ShapeDtypeStruct((n, 1), jnp.int32),
            jax.ShapeDtypeStruct((1, 1), jnp.float32),
        ],
        scratch_shapes=[pltpu.VMEM((_NUM_Q, 1, _K), jnp.float32)],
        interpret=interpret,
    )(z, codebooks)
    final_indices = jnp.concatenate([i0, i1, i2, i3], axis=1)
    total_loss = loss[0, 0] * ((1.0 + _COMMIT) / (n * d))
    return (zrec, final_indices, total_loss)


# speedup vs baseline: 1.0363x; 1.0363x over previous
import functools

import jax
import jax.numpy as jnp
from jax.experimental import pallas as pl
from jax.experimental.pallas import tpu as pltpu

_NUM_Q = 4
_K = 1024
_D = 64
_COMMIT = 0.25
_TILE = 4096
_CHAINS = 2

_PREC = jax.lax.Precision.DEFAULT
_PREC_WSQ = jax.lax.Precision.HIGHEST


def _rvq_body(z_ref, cb_ref, zrec_ref, i0_ref, i1_ref, i2_ref, i3_ref,
              loss_ref, wsq_ref):
    i = pl.program_id(0)

    @pl.when(i == 0)
    def _():
        for q in range(_NUM_Q):
            w = cb_ref[q]
            wsq_ref[q] = jax.lax.dot_general(
                jnp.ones((1, _D), jnp.float32), w * w,
                (((1,), (1,)), ((), ())), precision=_PREC_WSQ)

    z = z_ref[...]
    t = z.shape[0]
    nc = _CHAINS
    h = t // nc
    idx_refs = (i0_ref, i1_ref, i2_ref, i3_ref)
    lane_iota = jax.lax.broadcasted_iota(jnp.int32, (h, _K), 1)
    res = [z[c * h:(c + 1) * h] for c in range(nc)]
    zrec = [jnp.zeros_like(r) for r in res]
    loss_sum = jnp.float32(0.0)
    for q in range(_NUM_Q):
        w = cb_ref[q]
        wsq = wsq_ref[q]
        zw = [jax.lax.dot_general(res[c], w, (((1,), (1,)), ((), ())),
                                  precision=_PREC) for c in range(nc)]
        idx = [None] * nc
        one_hot = [None] * nc
        for c in range(nc):
            zsq = jnp.sum(res[c] * res[c], axis=1, keepdims=True)
            scores = (zsq + wsq) - 2.0 * zw[c]
            smin = jnp.min(scores, axis=1, keepdims=True)
            idx[c] = jnp.min(jnp.where(scores == smin, lane_iota, _K),
                             axis=1, keepdims=True)
            one_hot[c] = (lane_iota == idx[c]).astype(jnp.float32)
        for c in range(nc):
            idx_refs[q][pl.ds(c * h, h), :] = idx[c]
        for c in range(nc):
            zq = jax.lax.dot_general(
                one_hot[c], w, (((1,), (0,)), ((), ())),
                precision=_PREC)
            step = zq - res[c]
            zq_ste = res[c] + step
            loss_sum = loss_sum + jnp.sum(step * step)
            zrec[c] = zrec[c] + zq_ste
            res[c] = res[c] - zq_ste
    for c in range(nc):
        zrec_ref[pl.ds(c * h, h), :] = zrec[c]

    @pl.when(i == 0)
    def _():
        loss_ref[0, 0] = loss_sum

    @pl.when(i > 0)
    def _():
        loss_ref[0, 0] = loss_ref[0, 0] + loss_sum


@functools.partial(jax.jit, static_argnames=("interpret",))
def kernel(z, codebooks, interpret=False):
    n, d = z.shape
    grid = (n // _TILE,)
    idx_spec = pl.BlockSpec((_TILE, 1), lambda i: (i, 0))
    zrec, i0, i1, i2, i3, loss = pl.pallas_call(
        _rvq_body,
        grid=grid,
        in_specs=[
            pl.BlockSpec((_TILE, d), lambda i: (i, 0)),
            pl.BlockSpec((_NUM_Q, _K, _D), lambda i: (0, 0, 0)),
        ],
        out_specs=[
            pl.BlockSpec((_TILE, d), lambda i: (i, 0)),
            idx_spec, idx_spec, idx_spec, idx_spec,
            pl.BlockSpec((1, 1), lambda i: (0, 0),
                         memory_space=pltpu.SMEM),
        ],
        out_shape=[
            jax.ShapeDtypeStruct((n, d), jnp.float32),
            jax.ShapeDtypeStruct((n, 1), jnp.int32),
            jax.ShapeDtypeStruct((n, 1), jnp.int32),
            jax.ShapeDtypeStruct((n, 1), jnp.int32),
            jax.ShapeDtypeStruct((n, 1), jnp.int32),
            jax.ShapeDtypeStruct((1, 1), jnp.float32),
        ],
        scratch_shapes=[pltpu.VMEM((_NUM_Q, 1, _K), jnp.float32)],
        interpret=interpret,
    )(z, codebooks)
    final_indices = jnp.concatenate([i0, i1, i2, i3], axis=1)
    total_loss = loss[0, 0] * ((1.0 + _COMMIT) / (n * d))
    return (zrec, final_indices, total_loss)
